# baseline (speedup 1.0000x reference)
import functools

import jax
import jax.numpy as jnp
from jax import lax
from jax.experimental import pallas as pl
from jax.experimental.pallas import tpu as pltpu
from jax.experimental.pallas import tpu_sc as plsc

N = 10000
NP = 10240
E = 320000
EP = 327680
D = 128
NC, NS = 2, 16
NW = NC * NS
EPW = EP // NW
CHUNK = 128
NCHUNK = EPW // CHUNK
HCH = CHUNK // 2
RPT = NP // NS
NBLK = NP // 128

@functools.lru_cache(maxsize=1)
def _sc_kernels():
    mesh = plsc.VectorSubcoreMesh(core_axis_name="c", subcore_axis_name="s")

    @functools.partial(
        pl.kernel,
        out_type=jax.ShapeDtypeStruct((NW, NP), jnp.float32),
        mesh=mesh,
        compiler_params=pltpu.CompilerParams(needs_layout_passes=False),
        scratch_types=[
            pltpu.VMEM((EPW,), jnp.int32),
            pltpu.VMEM((NP,), jnp.float32),
        ],
    )
    def sc_degree(dst_flat, zeros_np, deg_out, idx_v, acc_v):
        w = lax.axis_index("c") * NS + lax.axis_index("s")
        pltpu.sync_copy(zeros_np, acc_v)
        pltpu.sync_copy(dst_flat.at[w], idx_v)
        ones = jnp.ones((16,), jnp.float32)

        def body(i, carry):
            idx = idx_v[pl.ds(i * 16, 16)]
            plsc.addupdate_scatter(acc_v, [idx], ones)
            return carry

        lax.fori_loop(0, EPW // 16, body, 0)
        pltpu.sync_copy(acc_v, deg_out.at[w])

    @functools.partial(
        pl.kernel,
        out_type=jax.ShapeDtypeStruct((NC, NP, D), jnp.float32),
        mesh=mesh,
        compiler_params=pltpu.CompilerParams(needs_layout_passes=False),
        scratch_types=[
            pltpu.VMEM((NCHUNK, CHUNK), jnp.int32),
            pltpu.VMEM((2 * NCHUNK, HCH), jnp.int32),
            pltpu.VMEM((HCH, D), jnp.float32),
            pltpu.VMEM((HCH, D), jnp.float32),
            pltpu.VMEM_SHARED((NP, D), jnp.float32),
            pltpu.SemaphoreType.DMA,
            pltpu.SemaphoreType.DMA,
            pltpu.SemaphoreType.DMA,
            pltpu.SemaphoreType.DMA,
        ],
    )
    def sc_aggregate(t_hbm, src_r, dst_r, zrows, agg_out,
                     src_v, dst_v, rows_a, rows_b, acc_sh,
                     sem_ga, sem_gb, sem_sa, sem_sb):
        c = lax.axis_index("c")
        s = lax.axis_index("s")
        w = c * NS + s
        pltpu.sync_copy(zrows, acc_sh.at[pl.ds(s * RPT, RPT)])
        pltpu.sync_copy(src_r.at[w], src_v)
        pltpu.sync_copy(dst_r.at[w], dst_v)
        plsc.subcore_barrier()

        def gather(row, half, rows, sem):
            idx = src_v.at[row, pl.ds(half * HCH, HCH)]
            return pltpu.async_copy(t_hbm.at[idx], rows, sem)

        def scatter(j, rows, sem):
            return pltpu.async_copy(rows, acc_sh.at[dst_v.at[j]], sem,
                                    add=True)

        def gwait(row, half, rows, sem):
            idx = src_v.at[row, pl.ds(half * HCH, HCH)]
            pltpu.make_async_copy(t_hbm.at[idx], rows, sem).wait()

        def swait(j, rows, sem):
            pltpu.make_async_copy(rows, acc_sh.at[dst_v.at[j]], sem).wait()

        gather(0, 0, rows_a, sem_ga)
        gwait(0, 0, rows_a, sem_ga)
        gather(0, 1, rows_b, sem_gb)
        scatter(0, rows_a, sem_sa)

        def body(k, carry):
            ja = 2 * k
            jb = 2 * k + 1
            gwait(k, 1, rows_b, sem_gb)
            swait(ja, rows_a, sem_sa)
            gather(k + 1, 0, rows_a, sem_ga)
            scatter(jb, rows_b, sem_sb)
            gwait(k + 1, 0, rows_a, sem_ga)
            swait(jb, rows_b, sem_sb)
            gather(k + 1, 1, rows_b, sem_gb)
            scatter(jb + 1, rows_a, sem_sa)
            return carry

        lax.fori_loop(0, NCHUNK - 1, body, 0)
        jl = 2 * NCHUNK - 1
        gwait(NCHUNK - 1, 1, rows_b, sem_gb)
        swait(jl - 1, rows_a, sem_sa)
        scatter(jl, rows_b, sem_sb)
        swait(jl, rows_b, sem_sb)
        plsc.subcore_barrier()
        pltpu.sync_copy(acc_sh.at[pl.ds(s * RPT, RPT)],
                        agg_out.at[c, pl.ds(s * RPT, RPT)])

    return sc_degree, sc_aggregate


def _dmat(dinv_row):
    ri = lax.broadcasted_iota(jnp.int32, (128, 128), 0)
    ci = lax.broadcasted_iota(jnp.int32, (128, 128), 1)
    return jnp.where(ri == ci, jnp.broadcast_to(dinv_row, (128, 128)), 0.0)


def _row_mask(b, h):
    rows = b * 128 + lax.broadcasted_iota(jnp.int32, (128, 128), 0)
    return jnp.where(rows < N, h, 0.0)


def _tc_layer1_body(x_ref, w1_ref, degp_ref, t1_ref, dinv_ref):
    deg = jnp.sum(degp_ref[...], axis=0, keepdims=True) + 1.0
    dinv = lax.rsqrt(deg)
    dinv_ref[...] = dinv[None]
    xw = jnp.dot(x_ref[...], w1_ref[...], preferred_element_type=jnp.float32)
    t1_ref[...] = jnp.dot(_dmat(dinv), xw, preferred_element_type=jnp.float32)


_tc_layer1 = pl.pallas_call(
    _tc_layer1_body,
    grid=(NBLK,),
    in_specs=[
        pl.BlockSpec((128, D), lambda b: (b, 0)),
        pl.BlockSpec((D, D), lambda b: (0, 0)),
        pl.BlockSpec((NW, 128), lambda b: (0, b)),
    ],
    out_specs=[
        pl.BlockSpec((128, D), lambda b: (b, 0)),
        pl.BlockSpec((1, 1, 128), lambda b: (b, 0, 0)),
    ],
    out_shape=[
        jax.ShapeDtypeStruct((NP, D), jnp.float32),
        jax.ShapeDtypeStruct((NBLK, 1, 128), jnp.float32),
    ],
)


def _tc_layer2_body(agg_ref, t1_ref, dinv_ref, w2_ref, b1_ref, t2_ref):
    b = pl.program_id(0)
    s = agg_ref[0] + agg_ref[1] + t1_ref[...]
    dm = _dmat(dinv_ref[0])
    pre = jnp.dot(dm, s, preferred_element_type=jnp.float32) + b1_ref[...]
    h = _row_mask(b, jnp.maximum(pre, 0.0))
    hw = jnp.dot(h, w2_ref[...], preferred_element_type=jnp.float32)
    t2_ref[...] = jnp.dot(dm, hw, preferred_element_type=jnp.float32)


_tc_layer2 = pl.pallas_call(
    _tc_layer2_body,
    grid=(NBLK,),
    in_specs=[
        pl.BlockSpec((NC, 128, D), lambda b: (0, b, 0)),
        pl.BlockSpec((128, D), lambda b: (b, 0)),
        pl.BlockSpec((1, 1, 128), lambda b: (b, 0, 0)),
        pl.BlockSpec((D, D), lambda b: (0, 0)),
        pl.BlockSpec((1, D), lambda b: (0, 0)),
    ],
    out_specs=pl.BlockSpec((128, D), lambda b: (b, 0)),
    out_shape=jax.ShapeDtypeStruct((NP, D), jnp.float32),
)


def _tc_final_body(agg_ref, t2_ref, dinv_ref, b2_ref, fcw_ref, fcb_ref,
                   out_ref, acc_ref):
    b = pl.program_id(0)

    @pl.when(b == 0)
    def _():
        acc_ref[...] = jnp.zeros_like(acc_ref)

    s = agg_ref[0] + agg_ref[1] + t2_ref[...]
    dm = _dmat(dinv_ref[0])
    pre = jnp.dot(dm, s, preferred_element_type=jnp.float32) + b2_ref[...]
    h = _row_mask(b, jnp.maximum(pre, 0.0))
    acc_ref[...] += jnp.sum(h, axis=0, keepdims=True)

    @pl.when(b == NBLK - 1)
    def _():
        pooled = acc_ref[...] * (1.0 / N)
        out_ref[...] = (jnp.sum(pooled * fcw_ref[...], axis=1, keepdims=True)
                        + fcb_ref[...])


_tc_final = pl.pallas_call(
    _tc_final_body,
    grid=(NBLK,),
    in_specs=[
        pl.BlockSpec((NC, 128, D), lambda b: (0, b, 0)),
        pl.BlockSpec((128, D), lambda b: (b, 0)),
        pl.BlockSpec((1, 1, 128), lambda b: (b, 0, 0)),
        pl.BlockSpec((1, D), lambda b: (0, 0)),
        pl.BlockSpec((1, D), lambda b: (0, 0)),
        pl.BlockSpec((1, 1), lambda b: (0, 0)),
    ],
    out_specs=pl.BlockSpec((1, 1), lambda b: (0, 0)),
    out_shape=jax.ShapeDtypeStruct((1, 1), jnp.float32),
    scratch_shapes=[pltpu.VMEM((1, 128), jnp.float32)],
)


def kernel(x, edge_index, edge_attr, W1, b1, W2, b2, edge_features, fc_w, fc_b):
    f32 = jnp.float32
    x_p = jnp.pad(x, ((0, NP - N), (0, 0)))
    pad_e = EP - E
    pad_cycle = N + (jnp.arange(pad_e, dtype=jnp.int32) % (NP - N))
    src_p = jnp.concatenate([edge_index[0].astype(jnp.int32), pad_cycle])
    dst_p = jnp.concatenate([edge_index[1].astype(jnp.int32), pad_cycle])
    src_r = src_p.reshape(NW, NCHUNK, CHUNK)
    dst_r = dst_p.reshape(NW, 2 * NCHUNK, HCH)
    dst_flat = dst_p.reshape(NW, EPW)
    zeros_np = jnp.zeros((NP,), f32)
    zrows = jnp.zeros((RPT, D), f32)

    sc_degree, sc_aggregate = _sc_kernels()
    deg_parts = sc_degree(dst_flat, zeros_np)
    t1, dinv = _tc_layer1(x_p, W1, deg_parts)
    agg1 = sc_aggregate(t1, src_r, dst_r, zrows)
    t2 = _tc_layer2(agg1, t1, dinv, W2, b1.reshape(1, D))
    agg2 = sc_aggregate(t2, src_r, dst_r, zrows)
    out = _tc_final(agg2, t2, dinv, b2.reshape(1, D),
                    fc_w.reshape(1, D), fc_b.reshape(1, 1))
    return out


# speedup vs baseline: 1.2246x; 1.2246x over previous
import functools

import jax
import jax.numpy as jnp
from jax import lax
from jax.experimental import pallas as pl
from jax.experimental.pallas import tpu as pltpu
from jax.experimental.pallas import tpu_sc as plsc

N = 10000
NP = 10240
E = 320000
EP = 327680
D = 128
NC, NS = 2, 16
NW = NC * NS
EPW = EP // NW
CHUNK = 128
NCHUNK = EPW // CHUNK
HCH = CHUNK // 2
RPT = NP // NS
NBLK = NP // 128

@functools.lru_cache(maxsize=1)
def _sc_kernels():
    mesh = plsc.VectorSubcoreMesh(core_axis_name="c", subcore_axis_name="s")

    @functools.partial(
        pl.kernel,
        out_type=jax.ShapeDtypeStruct((NW, NP), jnp.float32),
        mesh=mesh,
        compiler_params=pltpu.CompilerParams(needs_layout_passes=False),
        scratch_types=[
            pltpu.VMEM((EPW,), jnp.int32),
            pltpu.VMEM((NP,), jnp.float32),
        ],
    )
    def sc_degree(dst_flat, zeros_np, deg_out, idx_v, acc_v):
        w = lax.axis_index("c") * NS + lax.axis_index("s")
        pltpu.sync_copy(zeros_np, acc_v)
        pltpu.sync_copy(dst_flat.at[w], idx_v)
        ones = jnp.ones((16,), jnp.float32)

        def body(i, carry):
            idx = idx_v[pl.ds(i * 16, 16)]
            plsc.addupdate_scatter(acc_v, [idx], ones)
            return carry

        lax.fori_loop(0, EPW // 16, body, 0)
        pltpu.sync_copy(acc_v, deg_out.at[w])

    @functools.partial(
        pl.kernel,
        out_type=jax.ShapeDtypeStruct((NC, NP, D), jnp.float32),
        mesh=mesh,
        compiler_params=pltpu.CompilerParams(needs_layout_passes=False),
        scratch_types=[
            pltpu.VMEM((NCHUNK, CHUNK), jnp.int32),
            pltpu.VMEM((2 * NCHUNK, HCH), jnp.int32),
            pltpu.VMEM((HCH, D), jnp.float32),
            pltpu.VMEM((HCH, D), jnp.float32),
            pltpu.VMEM_SHARED((NP, D), jnp.float32),
            pltpu.SemaphoreType.DMA,
            pltpu.SemaphoreType.DMA,
            pltpu.SemaphoreType.DMA,
            pltpu.SemaphoreType.DMA,
        ],
    )
    def sc_aggregate(t_hbm, src_r, dst_r, zrows, agg_out,
                     src_v, dst_v, rows_a, rows_b, acc_sh,
                     sem_ga, sem_gb, sem_sa, sem_sb):
        c = lax.axis_index("c")
        s = lax.axis_index("s")
        w = c * NS + s
        pltpu.sync_copy(zrows, acc_sh.at[pl.ds(s * RPT, RPT)])
        pltpu.sync_copy(src_r.at[w], src_v)
        pltpu.sync_copy(dst_r.at[w], dst_v)
        plsc.subcore_barrier()

        def gather(row, half, rows, sem):
            idx = src_v.at[row, pl.ds(half * HCH, HCH)]
            return pltpu.async_copy(t_hbm.at[idx], rows, sem)

        def scatter(j, rows, sem):
            return pltpu.async_copy(rows, acc_sh.at[dst_v.at[j]], sem,
                                    add=True)

        def gwait(row, half, rows, sem):
            idx = src_v.at[row, pl.ds(half * HCH, HCH)]
            pltpu.make_async_copy(t_hbm.at[idx], rows, sem).wait()

        def swait(j, rows, sem):
            pltpu.make_async_copy(rows, acc_sh.at[dst_v.at[j]], sem).wait()

        gather(0, 0, rows_a, sem_ga)
        gwait(0, 0, rows_a, sem_ga)
        gather(0, 1, rows_b, sem_gb)
        scatter(0, rows_a, sem_sa)

        def body(k, carry):
            ja = 2 * k
            jb = 2 * k + 1
            gwait(k, 1, rows_b, sem_gb)
            swait(ja, rows_a, sem_sa)
            gather(k + 1, 0, rows_a, sem_ga)
            scatter(jb, rows_b, sem_sb)
            gwait(k + 1, 0, rows_a, sem_ga)
            swait(jb, rows_b, sem_sb)
            gather(k + 1, 1, rows_b, sem_gb)
            scatter(jb + 1, rows_a, sem_sa)
            return carry

        lax.fori_loop(0, NCHUNK - 1, body, 0)
        jl = 2 * NCHUNK - 1
        gwait(NCHUNK - 1, 1, rows_b, sem_gb)
        swait(jl - 1, rows_a, sem_sa)
        scatter(jl, rows_b, sem_sb)
        swait(jl, rows_b, sem_sb)
        plsc.subcore_barrier()
        pltpu.sync_copy(acc_sh.at[pl.ds(s * RPT, RPT)],
                        agg_out.at[c, pl.ds(s * RPT, RPT)])

    return sc_degree, sc_aggregate


BR = 512
NB2 = NP // BR


def _dcol(dinv_row):
    cols = []
    for k in range(BR // 128):
        piece = lax.slice(dinv_row, (0, k * 128), (1, (k + 1) * 128))
        ri = lax.broadcasted_iota(jnp.int32, (128, 128), 0)
        ci = lax.broadcasted_iota(jnp.int32, (128, 128), 1)
        dm = jnp.where(ri == ci, jnp.broadcast_to(piece, (128, 128)), 0.0)
        cols.append(jnp.sum(dm, axis=1, keepdims=True))
    return jnp.concatenate(cols, axis=0)


def _row_mask(b, h):
    rows = b * BR + lax.broadcasted_iota(jnp.int32, (BR, 128), 0)
    return jnp.where(rows < N, h, 0.0)


def _tc_layer1_body(x_ref, w1_ref, degp_ref, t1_ref, dinv_ref):
    deg = jnp.sum(degp_ref[...], axis=0, keepdims=True) + 1.0
    dinv = lax.rsqrt(deg)
    dinv_ref[...] = dinv[None]
    xw = jnp.dot(x_ref[...], w1_ref[...], preferred_element_type=jnp.float32)
    t1_ref[...] = xw * _dcol(dinv)


_tc_layer1 = pl.pallas_call(
    _tc_layer1_body,
    grid=(NB2,),
    in_specs=[
        pl.BlockSpec((BR, D), lambda b: (b, 0)),
        pl.BlockSpec((D, D), lambda b: (0, 0)),
        pl.BlockSpec((NW, BR), lambda b: (0, b)),
    ],
    out_specs=[
        pl.BlockSpec((BR, D), lambda b: (b, 0)),
        pl.BlockSpec((1, 1, BR), lambda b: (b, 0, 0)),
    ],
    out_shape=[
        jax.ShapeDtypeStruct((NP, D), jnp.float32),
        jax.ShapeDtypeStruct((NB2, 1, BR), jnp.float32),
    ],
)


def _tc_layer2_body(agg_ref, t1_ref, dinv_ref, w2_ref, b1_ref, t2_ref):
    b = pl.program_id(0)
    dc = _dcol(dinv_ref[0])
    s = agg_ref[0] + agg_ref[1] + t1_ref[...]
    pre = s * dc + b1_ref[...]
    h = _row_mask(b, jnp.maximum(pre, 0.0))
    hw = jnp.dot(h, w2_ref[...], preferred_element_type=jnp.float32)
    t2_ref[...] = hw * dc


_tc_layer2 = pl.pallas_call(
    _tc_layer2_body,
    grid=(NB2,),
    in_specs=[
        pl.BlockSpec((NC, BR, D), lambda b: (0, b, 0)),
        pl.BlockSpec((BR, D), lambda b: (b, 0)),
        pl.BlockSpec((1, 1, BR), lambda b: (b, 0, 0)),
        pl.BlockSpec((D, D), lambda b: (0, 0)),
        pl.BlockSpec((1, D), lambda b: (0, 0)),
    ],
    out_specs=pl.BlockSpec((BR, D), lambda b: (b, 0)),
    out_shape=jax.ShapeDtypeStruct((NP, D), jnp.float32),
)


def _tc_final_body(agg_ref, t2_ref, dinv_ref, b2_ref, fcw_ref, fcb_ref,
                   out_ref, acc_ref):
    b = pl.program_id(0)

    @pl.when(b == 0)
    def _():
        acc_ref[...] = jnp.zeros_like(acc_ref)

    dc = _dcol(dinv_ref[0])
    s = agg_ref[0] + agg_ref[1] + t2_ref[...]
    pre = s * dc + b2_ref[...]
    h = _row_mask(b, jnp.maximum(pre, 0.0))
    acc_ref[...] += jnp.sum(h, axis=0, keepdims=True)

    @pl.when(b == NB2 - 1)
    def _():
        pooled = acc_ref[...] * (1.0 / N)
        out_ref[...] = (jnp.sum(pooled * fcw_ref[...], axis=1, keepdims=True)
                        + fcb_ref[...])


_tc_final = pl.pallas_call(
    _tc_final_body,
    grid=(NB2,),
    in_specs=[
        pl.BlockSpec((NC, BR, D), lambda b: (0, b, 0)),
        pl.BlockSpec((BR, D), lambda b: (b, 0)),
        pl.BlockSpec((1, 1, BR), lambda b: (b, 0, 0)),
        pl.BlockSpec((1, D), lambda b: (0, 0)),
        pl.BlockSpec((1, D), lambda b: (0, 0)),
        pl.BlockSpec((1, 1), lambda b: (0, 0)),
    ],
    out_specs=pl.BlockSpec((1, 1), lambda b: (0, 0)),
    out_shape=jax.ShapeDtypeStruct((1, 1), jnp.float32),
    scratch_shapes=[pltpu.VMEM((1, 128), jnp.float32)],
)


def kernel(x, edge_index, edge_attr, W1, b1, W2, b2, edge_features, fc_w, fc_b):
    f32 = jnp.float32
    x_p = jnp.pad(x, ((0, NP - N), (0, 0)))
    pad_e = EP - E
    pad_cycle = N + (jnp.arange(pad_e, dtype=jnp.int32) % (NP - N))
    src_p = jnp.concatenate([edge_index[0].astype(jnp.int32), pad_cycle])
    dst_p = jnp.concatenate([edge_index[1].astype(jnp.int32), pad_cycle])
    src_r = src_p.reshape(NW, NCHUNK, CHUNK)
    dst_r = dst_p.reshape(NW, 2 * NCHUNK, HCH)
    dst_flat = dst_p.reshape(NW, EPW)
    zeros_np = jnp.zeros((NP,), f32)
    zrows = jnp.zeros((RPT, D), f32)

    sc_degree, sc_aggregate = _sc_kernels()
    deg_parts = sc_degree(dst_flat, zeros_np)
    t1, dinv = _tc_layer1(x_p, W1, deg_parts)
    agg1 = sc_aggregate(t1, src_r, dst_r, zrows)
    t2 = _tc_layer2(agg1, t1, dinv, W2, b1.reshape(1, D))
    agg2 = sc_aggregate(t2, src_r, dst_r, zrows)
    out = _tc_final(agg2, t2, dinv, b2.reshape(1, D),
                    fc_w.reshape(1, D), fc_b.reshape(1, 1))
    return out


# speedup vs baseline: 1.2688x; 1.0361x over previous
import functools

import jax
import jax.numpy as jnp
from jax import lax
from jax.experimental import pallas as pl
from jax.experimental.pallas import tpu as pltpu
from jax.experimental.pallas import tpu_sc as plsc

N = 10000
NP = 10240
E = 320000
EP = 327680
D = 128
NC, NS = 2, 16
NW = NC * NS
EPW = EP // NW
CHUNK = 128
NCHUNK = EPW // CHUNK
HCH = CHUNK // 2
RPT = NP // NS
NBLK = NP // 128

@functools.lru_cache(maxsize=1)
def _sc_kernels():
    mesh = plsc.VectorSubcoreMesh(core_axis_name="c", subcore_axis_name="s")

    @functools.partial(
        pl.kernel,
        out_type=jax.ShapeDtypeStruct((NW, NP), jnp.float32),
        mesh=mesh,
        compiler_params=pltpu.CompilerParams(needs_layout_passes=False),
        scratch_types=[
            pltpu.VMEM((EPW,), jnp.int32),
            pltpu.VMEM((NP,), jnp.float32),
        ],
    )
    def sc_degree(dst_flat, zeros_np, deg_out, idx_v, acc_v):
        w = lax.axis_index("c") * NS + lax.axis_index("s")
        pltpu.sync_copy(zeros_np, acc_v)
        pltpu.sync_copy(dst_flat.at[w], idx_v)
        ones = jnp.ones((16,), jnp.float32)

        def body(i, carry):
            idx = idx_v[pl.ds(i * 16, 16)]
            plsc.addupdate_scatter(acc_v, [idx], ones)
            return carry

        lax.fori_loop(0, EPW // 16, body, 0)
        pltpu.sync_copy(acc_v, deg_out.at[w])

    @functools.partial(
        pl.kernel,
        out_type=jax.ShapeDtypeStruct((NC, NP, D), jnp.float32),
        mesh=mesh,
        compiler_params=pltpu.CompilerParams(needs_layout_passes=False),
        scratch_types=[
            pltpu.VMEM((NCHUNK, CHUNK), jnp.int32),
            pltpu.VMEM((2 * NCHUNK, HCH), jnp.int32),
            pltpu.VMEM((HCH, D), jnp.float32),
            pltpu.VMEM((HCH, D), jnp.float32),
            pltpu.VMEM_SHARED((NP, D), jnp.float32),
            pltpu.SemaphoreType.DMA,
            pltpu.SemaphoreType.DMA,
            pltpu.SemaphoreType.DMA,
            pltpu.SemaphoreType.DMA,
        ],
    )
    def sc_aggregate(t_hbm, src_r, dst_r, zrows, agg_out,
                     src_v, dst_v, rows_a, rows_b, acc_sh,
                     sem_ga, sem_gb, sem_sa, sem_sb):
        c = lax.axis_index("c")
        s = lax.axis_index("s")
        w = c * NS + s
        pltpu.sync_copy(zrows, acc_sh.at[pl.ds(s * RPT, RPT)])
        pltpu.sync_copy(src_r.at[w], src_v)
        pltpu.sync_copy(dst_r.at[w], dst_v)
        plsc.subcore_barrier()

        def gather(row, half, rows, sem):
            idx = src_v.at[row, pl.ds(half * HCH, HCH)]
            return pltpu.async_copy(t_hbm.at[idx], rows, sem)

        def scatter(j, rows, sem):
            return pltpu.async_copy(rows, acc_sh.at[dst_v.at[j]], sem,
                                    add=True)

        def gwait(row, half, rows, sem):
            idx = src_v.at[row, pl.ds(half * HCH, HCH)]
            pltpu.make_async_copy(t_hbm.at[idx], rows, sem).wait()

        def swait(j, rows, sem):
            pltpu.make_async_copy(rows, acc_sh.at[dst_v.at[j]], sem).wait()

        gather(0, 0, rows_a, sem_ga)
        gwait(0, 0, rows_a, sem_ga)
        gather(0, 1, rows_b, sem_gb)
        scatter(0, rows_a, sem_sa)

        def body(k, carry):
            ja = 2 * k
            jb = 2 * k + 1
            gwait(k, 1, rows_b, sem_gb)
            swait(ja, rows_a, sem_sa)
            gather(k + 1, 0, rows_a, sem_ga)
            scatter(jb, rows_b, sem_sb)
            gwait(k + 1, 0, rows_a, sem_ga)
            swait(jb, rows_b, sem_sb)
            gather(k + 1, 1, rows_b, sem_gb)
            scatter(jb + 1, rows_a, sem_sa)
            return carry

        lax.fori_loop(0, NCHUNK - 1, body, 0)
        jl = 2 * NCHUNK - 1
        gwait(NCHUNK - 1, 1, rows_b, sem_gb)
        swait(jl - 1, rows_a, sem_sa)
        scatter(jl, rows_b, sem_sb)
        swait(jl, rows_b, sem_sb)
        plsc.subcore_barrier()
        pltpu.sync_copy(acc_sh.at[pl.ds(s * RPT, RPT)],
                        agg_out.at[c, pl.ds(s * RPT, RPT)])

    return sc_degree, sc_aggregate


BR = 1024
NB2 = NP // BR


def _dcol(dinv_row):
    cols = []
    for k in range(BR // 128):
        piece = lax.slice(dinv_row, (0, k * 128), (1, (k + 1) * 128))
        ri = lax.broadcasted_iota(jnp.int32, (128, 128), 0)
        ci = lax.broadcasted_iota(jnp.int32, (128, 128), 1)
        dm = jnp.where(ri == ci, jnp.broadcast_to(piece, (128, 128)), 0.0)
        cols.append(jnp.sum(dm, axis=1, keepdims=True))
    return jnp.concatenate(cols, axis=0)


def _row_mask(b, h):
    rows = b * BR + lax.broadcasted_iota(jnp.int32, (BR, 128), 0)
    return jnp.where(rows < N, h, 0.0)


def _tc_layer1_body(x_ref, w1_ref, degp_ref, t1_ref, dinv_ref):
    deg = jnp.sum(degp_ref[...], axis=0, keepdims=True) + 1.0
    dinv = lax.rsqrt(deg)
    dinv_ref[...] = dinv[None]
    xw = jnp.dot(x_ref[...], w1_ref[...], preferred_element_type=jnp.float32)
    t1_ref[...] = xw * _dcol(dinv)


_tc_layer1 = pl.pallas_call(
    _tc_layer1_body,
    grid=(NB2,),
    in_specs=[
        pl.BlockSpec((BR, D), lambda b: (b, 0)),
        pl.BlockSpec((D, D), lambda b: (0, 0)),
        pl.BlockSpec((NW, BR), lambda b: (0, b)),
    ],
    out_specs=[
        pl.BlockSpec((BR, D), lambda b: (b, 0)),
        pl.BlockSpec((1, 1, BR), lambda b: (b, 0, 0)),
    ],
    out_shape=[
        jax.ShapeDtypeStruct((NP, D), jnp.float32),
        jax.ShapeDtypeStruct((NB2, 1, BR), jnp.float32),
    ],
)


def _tc_layer2_body(agg_ref, t1_ref, dinv_ref, w2_ref, b1_ref, t2_ref):
    b = pl.program_id(0)
    dc = _dcol(dinv_ref[0])
    s = agg_ref[0] + agg_ref[1] + t1_ref[...]
    pre = s * dc + b1_ref[...]
    h = _row_mask(b, jnp.maximum(pre, 0.0))
    hw = jnp.dot(h, w2_ref[...], preferred_element_type=jnp.float32)
    t2_ref[...] = hw * dc


_tc_layer2 = pl.pallas_call(
    _tc_layer2_body,
    grid=(NB2,),
    in_specs=[
        pl.BlockSpec((NC, BR, D), lambda b: (0, b, 0)),
        pl.BlockSpec((BR, D), lambda b: (b, 0)),
        pl.BlockSpec((1, 1, BR), lambda b: (b, 0, 0)),
        pl.BlockSpec((D, D), lambda b: (0, 0)),
        pl.BlockSpec((1, D), lambda b: (0, 0)),
    ],
    out_specs=pl.BlockSpec((BR, D), lambda b: (b, 0)),
    out_shape=jax.ShapeDtypeStruct((NP, D), jnp.float32),
)


def _tc_final_body(agg_ref, t2_ref, dinv_ref, b2_ref, fcw_ref, fcb_ref,
                   out_ref, acc_ref):
    b = pl.program_id(0)

    @pl.when(b == 0)
    def _():
        acc_ref[...] = jnp.zeros_like(acc_ref)

    dc = _dcol(dinv_ref[0])
    s = agg_ref[0] + agg_ref[1] + t2_ref[...]
    pre = s * dc + b2_ref[...]
    h = _row_mask(b, jnp.maximum(pre, 0.0))
    acc_ref[...] += jnp.sum(h, axis=0, keepdims=True)

    @pl.when(b == NB2 - 1)
    def _():
        pooled = acc_ref[...] * (1.0 / N)
        out_ref[...] = (jnp.sum(pooled * fcw_ref[...], axis=1, keepdims=True)
                        + fcb_ref[...])


_tc_final = pl.pallas_call(
    _tc_final_body,
    grid=(NB2,),
    in_specs=[
        pl.BlockSpec((NC, BR, D), lambda b: (0, b, 0)),
        pl.BlockSpec((BR, D), lambda b: (b, 0)),
        pl.BlockSpec((1, 1, BR), lambda b: (b, 0, 0)),
        pl.BlockSpec((1, D), lambda b: (0, 0)),
        pl.BlockSpec((1, D), lambda b: (0, 0)),
        pl.BlockSpec((1, 1), lambda b: (0, 0)),
    ],
    out_specs=pl.BlockSpec((1, 1), lambda b: (0, 0)),
    out_shape=jax.ShapeDtypeStruct((1, 1), jnp.float32),
    scratch_shapes=[pltpu.VMEM((1, 128), jnp.float32)],
)


def kernel(x, edge_index, edge_attr, W1, b1, W2, b2, edge_features, fc_w, fc_b):
    f32 = jnp.float32
    x_p = jnp.pad(x, ((0, NP - N), (0, 0)))
    pad_e = EP - E
    pad_cycle = N + (jnp.arange(pad_e, dtype=jnp.int32) % (NP - N))
    src_p = jnp.concatenate([edge_index[0].astype(jnp.int32), pad_cycle])
    dst_p = jnp.concatenate([edge_index[1].astype(jnp.int32), pad_cycle])
    src_r = src_p.reshape(NW, NCHUNK, CHUNK)
    dst_r = dst_p.reshape(NW, 2 * NCHUNK, HCH)
    dst_flat = dst_p.reshape(NW, EPW)
    zeros_np = jnp.zeros((NP,), f32)
    zrows = jnp.zeros((RPT, D), f32)

    sc_degree, sc_aggregate = _sc_kernels()
    deg_parts = sc_degree(dst_flat, zeros_np)
    t1, dinv = _tc_layer1(x_p, W1, deg_parts)
    agg1 = sc_aggregate(t1, src_r, dst_r, zrows)
    t2 = _tc_layer2(agg1, t1, dinv, W2, b1.reshape(1, D))
    agg2 = sc_aggregate(t2, src_r, dst_r, zrows)
    out = _tc_final(agg2, t2, dinv, b2.reshape(1, D),
                    fc_w.reshape(1, D), fc_b.reshape(1, 1))
    return out
